# Initial kernel scaffold; baseline (speedup 1.0000x reference)
import jax
import jax.numpy as jnp
from jax.experimental import pallas as pl


def kernel(x, W1, b1, W2, b2, W3, b3):
    raise NotImplementedError("write your pallas kernel here")


# speedup vs baseline: 18.9930x; 18.9930x over previous
import functools

import jax
import jax.numpy as jnp
from jax.experimental import pallas as pl

NEG = -3.0e38


def _pixel_unshuffle(x, r):
    B, C, H, W = x.shape
    x = x.reshape(B, C, H // r, r, W // r, r)
    x = x.transpose(0, 1, 3, 5, 2, 4)
    return x.reshape(B, C * r * r, H // r, W // r)


def _pixel_shuffle(x, r):
    B, C, H, W = x.shape
    x = x.reshape(B, C // (r * r), r, r, H, W)
    x = x.transpose(0, 1, 4, 2, 5, 3)
    return x.reshape(B, C // (r * r), H * r, W * r)


def _conv1_body(xp_ref, w_ref, b_ref, o_ref):
    acc = jnp.zeros((16, 128 * 128), jnp.float32)
    for dy in range(3):
        for dx in range(3):
            xs = xp_ref[0, :, dy:dy + 128, dx:dx + 128].reshape(3, 128 * 128)
            acc = acc + jax.lax.dot_general(
                w_ref[:, :, dy, dx], xs, (((1,), (0,)), ((), ())),
                preferred_element_type=jnp.float32)
    acc = jnp.maximum(acc + b_ref[:], 0.0)
    o_ref[0] = acc.reshape(16, 128, 128)


def _conv1(x, W1, b1):
    B = x.shape[0]
    xp = jnp.pad(x, ((0, 0), (0, 0), (1, 1), (1, 1)))
    return pl.pallas_call(
        _conv1_body,
        grid=(B,),
        in_specs=[
            pl.BlockSpec((1, 3, 130, 130), lambda b: (b, 0, 0, 0)),
            pl.BlockSpec((16, 3, 3, 3), lambda b: (0, 0, 0, 0)),
            pl.BlockSpec((16, 1), lambda b: (0, 0)),
        ],
        out_specs=pl.BlockSpec((1, 16, 128, 128), lambda b: (b, 0, 0, 0)),
        out_shape=jax.ShapeDtypeStruct((B, 16, 128, 128), jnp.float32),
    )(xp, W1, b1.reshape(16, 1))


def _knn_body(xf_ref, xr_ref, wr_ref, b_ref, o_ref, *, RB, K, relu):
    x = xf_ref[0]
    C, N = x.shape
    nrm = jnp.sqrt(jnp.sum(x * x, axis=0, keepdims=True))
    xn = x * (1.0 / (nrm + 1e-8))
    xr = xr_ref[0]
    rnrm = jnp.sqrt(jnp.sum(xr * xr, axis=0, keepdims=True))
    rows = xr * (1.0 / (rnrm + 1e-8))
    sim = jax.lax.dot_general(rows, xn, (((0,), (0,)), ((), ())),
                              preferred_element_type=jnp.float32)
    iota = jax.lax.broadcasted_iota(jnp.int32, (RB, N), 1)
    gs = []
    for _ in range(K):
        mx = jnp.max(sim, axis=1, keepdims=True)
        idxk = jnp.min(jnp.where(sim == mx, iota, N), axis=1, keepdims=True)
        hit = iota == idxk
        oh = hit.astype(jnp.float32)
        gs.append(jax.lax.dot_general(oh, x, (((1,), (1,)), ((), ())),
                                      preferred_element_type=jnp.float32))
        sim = jnp.where(hit, NEG, sim)
    G = jnp.concatenate(gs, axis=1)
    out = jax.lax.dot_general(G, wr_ref[:], (((1,), (0,)), ((), ())),
                              preferred_element_type=jnp.float32)
    out = out + b_ref[:]
    if relu:
        out = jnp.maximum(out, 0.0)
    o_ref[0] = out


def _conv_nn(h, W, b, K, r, relu):
    xu = _pixel_unshuffle(h, r)
    B, C, Hh, Ww = xu.shape
    N = Hh * Ww
    O = W.shape[0]
    xf = xu.reshape(B, C, N)
    wr = W.transpose(2, 1, 0).reshape(K * C, O)
    RB = 256
    body = functools.partial(_knn_body, RB=RB, K=K, relu=relu)
    out = pl.pallas_call(
        body,
        grid=(B, N // RB),
        in_specs=[
            pl.BlockSpec((1, C, N), lambda b, i: (b, 0, 0)),
            pl.BlockSpec((1, C, RB), lambda b, i: (b, 0, i)),
            pl.BlockSpec((K * C, O), lambda b, i: (0, 0)),
            pl.BlockSpec((1, O), lambda b, i: (0, 0)),
        ],
        out_specs=pl.BlockSpec((1, RB, O), lambda b, i: (b, i, 0)),
        out_shape=jax.ShapeDtypeStruct((B, N, O), jnp.float32),
    )(xf, xf, wr, b.reshape(1, O))
    out = out.transpose(0, 2, 1).reshape(B, O, Hh, Ww)
    return _pixel_shuffle(out, r)


def kernel(x, W1, b1, W2, b2, W3, b3):
    h = _conv1(x, W1, b1)
    h = _conv_nn(h, W2, b2, 9, 2, relu=True)
    return _conv_nn(h, W3, b3, 9, 2, relu=False)
